# baseline (speedup 1.0000x reference)
import functools

import jax
import jax.numpy as jnp
from jax import lax
from jax.experimental import pallas as pl
from jax.experimental.pallas import tpu as pltpu
from jax.experimental.pallas import tpu_sc as plsc

VOCAB = 1000000
D_MODEL = 64
SEQ = 200
BATCH = 1024
N_FLAT = BATCH * SEQ

CHUNK = 128


def _make_kernel():
    info = plsc.get_sparse_core_info()
    nc, ns = info.num_cores, info.num_subcores
    nw = nc * ns
    per_w = N_FLAT // nw
    assert N_FLAT % nw == 0 and per_w % CHUNK == 0
    n_chunks = per_w // CHUNK

    mesh = plsc.VectorSubcoreMesh(core_axis_name="c", subcore_axis_name="s")

    @functools.partial(
        pl.kernel,
        mesh=mesh,
        out_type=jax.ShapeDtypeStruct((N_FLAT, D_MODEL), jnp.float32),
        scratch_types=[
            pltpu.VMEM((per_w,), jnp.int32),
            pltpu.VMEM((SEQ, D_MODEL), jnp.float32),
            pltpu.VMEM((CHUNK, D_MODEL), jnp.float32),
            pltpu.SemaphoreType.DMA,
        ],
        compiler_params=pltpu.CompilerParams(use_tc_tiling_on_sc=False),
    )
    def emb_kernel(x_hbm, tab_hbm, pos_hbm, out_hbm, idx_v, pos_v, buf_v, sem):
        wid = lax.axis_index("s") * nc + lax.axis_index("c")
        base = wid * per_w
        pltpu.sync_copy(x_hbm.at[pl.ds(base, per_w)], idx_v)
        pltpu.sync_copy(pos_hbm, pos_v)

        def chunk_body(c, carry):
            off = pl.multiple_of(c * CHUNK, CHUNK)
            pltpu.async_copy(
                tab_hbm.at[idx_v.at[pl.ds(off, CHUNK)]], buf_v, sem
            ).wait()

            def row_body(r, carry2):
                t = lax.rem(off + r, SEQ)
                for j in range(D_MODEL // 16):
                    sl = pl.ds(j * 16, 16)
                    buf_v[r, sl] = buf_v[r, sl] + pos_v[t, sl]
                return carry2

            lax.fori_loop(0, CHUNK, row_body, 0, unroll=False)
            pltpu.sync_copy(buf_v, out_hbm.at[pl.ds(base + off, CHUNK)])
            return carry

        lax.fori_loop(0, n_chunks, chunk_body, 0, unroll=False)

    return emb_kernel


_emb_kernel = _make_kernel()


@jax.jit
def kernel(x, token_table, pos_embed):
    seq = x.shape[1]
    x_flat = x.reshape(-1).astype(jnp.int32)
    pos = pos_embed[0, :seq, :].astype(jnp.float32)
    out_flat = _emb_kernel(x_flat, token_table, pos)
    return out_flat.reshape(x.shape[0], seq, D_MODEL)


# speedup vs baseline: 1.0514x; 1.0514x over previous
import functools

import jax
import jax.numpy as jnp
from jax import lax
from jax.experimental import pallas as pl
from jax.experimental.pallas import tpu as pltpu
from jax.experimental.pallas import tpu_sc as plsc

VOCAB = 1000000
D_MODEL = 64
SEQ = 200
BATCH = 1024
N_FLAT = BATCH * SEQ

CHUNK = 128
NBUF = 5


def _make_kernel():
    info = plsc.get_sparse_core_info()
    nc, ns = info.num_cores, info.num_subcores
    nw = nc * ns
    per_w = N_FLAT // nw
    assert N_FLAT % nw == 0 and per_w % CHUNK == 0
    n_chunks = per_w // CHUNK
    assert n_chunks % NBUF == 0
    n_outer = n_chunks // NBUF

    mesh = plsc.VectorSubcoreMesh(core_axis_name="c", subcore_axis_name="s")

    @functools.partial(
        pl.kernel,
        mesh=mesh,
        out_type=jax.ShapeDtypeStruct((N_FLAT, D_MODEL), jnp.float32),
        scratch_types=[
            pltpu.VMEM((per_w,), jnp.int32),
            pltpu.VMEM((2 * SEQ, D_MODEL), jnp.float32),
            [pltpu.VMEM((CHUNK, D_MODEL), jnp.float32) for _ in range(NBUF)],
            [pltpu.SemaphoreType.DMA for _ in range(NBUF)],
            [pltpu.SemaphoreType.DMA for _ in range(NBUF)],
        ],
        compiler_params=pltpu.CompilerParams(use_tc_tiling_on_sc=False),
    )
    def emb_kernel(x_hbm, tab_hbm, pos2_hbm, out_hbm, idx_v, pos_v, bufs,
                   gsems, ssems):
        wid = lax.axis_index("s") * nc + lax.axis_index("c")
        base = wid * per_w
        pltpu.sync_copy(x_hbm.at[pl.ds(base, per_w)], idx_v)
        pltpu.sync_copy(pos2_hbm, pos_v)

        def gather_start(c, b):
            off = pl.multiple_of(c * CHUNK, CHUNK)
            pltpu.async_copy(
                tab_hbm.at[idx_v.at[pl.ds(off, CHUNK)]], bufs[b], gsems[b]
            )

        def gather_wait(c, b):
            pltpu.make_async_copy(
                tab_hbm.at[idx_v.at[pl.ds(0, CHUNK)]], bufs[b], gsems[b]
            ).wait()

        def store_start(c, b):
            off = pl.multiple_of(c * CHUNK, CHUNK)
            pltpu.async_copy(bufs[b], out_hbm.at[pl.ds(base + off, CHUNK)],
                             ssems[b])

        def store_wait(b):
            pltpu.make_async_copy(
                bufs[b], out_hbm.at[pl.ds(base, CHUNK)], ssems[b]
            ).wait()

        def compute(c, b):
            t0 = lax.rem(c * CHUNK, SEQ)
            buf = bufs[b]

            def row_body(r, carry):
                for u in range(2):
                    rr = r * 2 + u
                    t = t0 + rr
                    for j in range(D_MODEL // 16):
                        sl = pl.ds(j * 16, 16)
                        buf[rr, sl] = buf[rr, sl] + pos_v[t, sl]
                return carry

            lax.fori_loop(0, CHUNK // 2, row_body, 0, unroll=False)

        for b in range(NBUF - 1):
            gather_start(b, b)

        def outer(c0, carry):
            for b in range(NBUF):
                c = c0 * NBUF + b
                gather_wait(c, b)
                compute(c, b)
                store_start(c, b)
                bg = (b + NBUF - 1) % NBUF
                g = c + NBUF - 1
                if b == 0:
                    @pl.when(c0 >= 1)
                    def _():
                        store_wait(bg)
                    @pl.when(c0 * NBUF + NBUF - 1 < n_chunks)
                    def _():
                        gather_start(g, bg)
                else:
                    store_wait(bg)

                    @pl.when(g < n_chunks)
                    def _():
                        gather_start(g, bg)
            return carry

        lax.fori_loop(0, n_outer, outer, 0, unroll=False)
        store_wait((n_chunks - 1) % NBUF)

    return emb_kernel


_emb_kernel = _make_kernel()


@jax.jit
def kernel(x, token_table, pos_embed):
    seq = x.shape[1]
    x_flat = x.reshape(-1).astype(jnp.int32)
    pos = pos_embed[0, :seq, :].astype(jnp.float32)
    pos2 = jnp.concatenate([pos, pos], axis=0)
    out_flat = _emb_kernel(x_flat, token_table, pos2)
    return out_flat.reshape(x.shape[0], seq, D_MODEL)
